# baseline (device time: 68774 ns/iter reference)
import jax
import jax.numpy as jnp
from jax import lax
from jax.experimental import pallas as pl
from jax.experimental.pallas import tpu as pltpu

N_DEV = 8
M_BLK = 512
K_BLK = 512
KB = 1024
NB = 2048
K_STEPS = 4
N_STEPS = 4


def kernel(x, w_mat):
    m_tot, k_loc = x.shape
    k_tot, n = w_mat.shape

    def body(x_ref, w_ref, out_ref, gath_ref):
        tn = pl.program_id(0)
        tk = pl.program_id(1)

        @pl.when((tn == 0) & (tk == 0))
        def _first_step():
            for d in range(N_DEV):
                gath_ref[:, pl.ds(d * K_BLK, K_BLK)] = x_ref[
                    pl.ds(d * M_BLK, M_BLK), :
                ].astype(jnp.bfloat16)

        acc = jnp.dot(
            gath_ref[:, pl.ds(tk * KB, KB)],
            w_ref[...].astype(jnp.bfloat16),
            preferred_element_type=jnp.float32,
        )

        @pl.when(tk == 0)
        def _init():
            out_ref[...] = acc

        @pl.when(tk != 0)
        def _accum():
            out_ref[...] += acc

        @pl.when(tk == K_STEPS - 1)
        def _epilogue():
            y = out_ref[...]
            out_ref[...] = y * jax.nn.sigmoid(y)

    return pl.pallas_call(
        body,
        grid=(N_STEPS, K_STEPS),
        in_specs=[
            pl.BlockSpec((m_tot, K_BLK), lambda tn, tk: (0, 0)),
            pl.BlockSpec((KB, NB), lambda tn, tk: (tk, tn)),
        ],
        out_specs=pl.BlockSpec((M_BLK, NB), lambda tn, tk: (0, tn)),
        out_shape=jax.ShapeDtypeStruct((M_BLK, n), jnp.float32),
        scratch_shapes=[
            pltpu.VMEM((M_BLK, k_tot), jnp.bfloat16),
        ],
        compiler_params=pltpu.CompilerParams(
            dimension_semantics=("arbitrary", "arbitrary"),
        ),
    )(x, w_mat)
